# Initial kernel scaffold; baseline (speedup 1.0000x reference)
import jax
import jax.numpy as jnp
from jax.experimental import pallas as pl


def kernel(z, u, table):
    raise NotImplementedError("write your pallas kernel here")


# speedup vs baseline: 1.1458x; 1.1458x over previous
import functools

import jax
import jax.numpy as jnp
from jax.experimental import pallas as pl
from jax.experimental.pallas import tpu as pltpu
from jax.experimental.pallas import tpu_sc as plsc

_NC, _NS, _L = 2, 16, 16
_W = 128


def kernel(z, u, table):
    B, S = z.shape
    D = table.shape[1]
    N = B * S
    idx = z.reshape(1, N).astype(jnp.int32)
    u2 = u.reshape(N, D)

    mesh = plsc.VectorSubcoreMesh(core_axis_name="c", subcore_axis_name="s")

    @functools.partial(
        pl.kernel,
        out_type=jax.ShapeDtypeStruct((N, D), jnp.float32),
        mesh=mesh,
        compiler_params=pltpu.CompilerParams(use_tc_tiling_on_sc=False),
    )
    def run(table_hbm, idx_hbm, u_hbm, o_hbm):
        def body(i_vmem, u_vmem, o_vmem):
            for c in range(0, _W, _L):
                i_vmem[0, pl.ds(c, _L)] = i_vmem[0, pl.ds(c, _L)] + 1
            pltpu.sync_copy(table_hbm.at[i_vmem.at[0]], o_vmem)

            @pl.loop(0, _W)
            def _(r):
                for c in range(0, D, _L):
                    o_vmem[r, pl.ds(c, _L)] = (
                        o_vmem[r, pl.ds(c, _L)] + u_vmem[r, pl.ds(c, _L)]
                    )

        pltpu.emit_pipeline(
            body,
            grid=(N // _W,),
            in_specs=[
                pl.BlockSpec((1, _W), lambda i: (0, i)),
                pl.BlockSpec((_W, D), lambda i: (i, 0)),
            ],
            out_specs=[pl.BlockSpec((_W, D), lambda i: (i, 0))],
            core_axis_name=("c", "s"),
            dimension_semantics=(pltpu.PARALLEL,),
        )(idx_hbm, u_hbm, o_hbm)

    v = run(table, idx, u2)
    return (z, v.reshape(B, S, D))


# speedup vs baseline: 1.5494x; 1.3522x over previous
import functools

import jax
import jax.numpy as jnp
from jax import lax
from jax.experimental import pallas as pl
from jax.experimental.pallas import tpu as pltpu
from jax.experimental.pallas import tpu_sc as plsc

_NC, _NS, _L = 2, 16, 16
_NW = _NC * _NS
_C = 256
_R = 4


def kernel(z, u, table):
    B, S = z.shape
    D = table.shape[1]
    N = B * S
    n_chunks = N // (_NW * _C)

    idx = z.reshape(N).astype(jnp.int32)
    u2 = u.reshape(N, D)

    mesh = plsc.VectorSubcoreMesh(core_axis_name="c", subcore_axis_name="s")

    @functools.partial(
        pl.kernel,
        out_type=jax.ShapeDtypeStruct((N, D), jnp.float32),
        mesh=mesh,
        compiler_params=pltpu.CompilerParams(use_tc_tiling_on_sc=False),
        scratch_types=[
            pltpu.VMEM((_R, _C), jnp.int32),
            pltpu.VMEM((_R, _C, 64), jnp.float32),
            pltpu.SemaphoreType.DMA((_R,)),
            pltpu.SemaphoreType.DMA((_R,)),
            pltpu.SemaphoreType.DMA((_R,)),
            pltpu.SemaphoreType.DMA((_R,)),
        ],
    )
    def run(table_hbm, idx_hbm, u_hbm, o_hbm, idx_v, acc_v, s_i, s_u, s_g, s_o):
        wid = lax.axis_index("s") * _NC + lax.axis_index("c")
        base = wid * (n_chunks * _C)

        def start_inputs(i, p):
            off = base + i * _C
            pltpu.async_copy(idx_hbm.at[pl.ds(off, _C)], idx_v.at[p], s_i.at[p])
            pltpu.async_copy(u_hbm.at[pl.ds(off, _C)], acc_v.at[p], s_u.at[p])

        def fire_gather(i, p):
            pltpu.make_async_copy(idx_hbm.at[pl.ds(0, _C)], idx_v.at[p],
                                  s_i.at[p]).wait()
            pltpu.make_async_copy(u_hbm.at[pl.ds(0, _C)], acc_v.at[p],
                                  s_u.at[p]).wait()
            for c in range(0, _C, _L):
                idx_v[p, pl.ds(c, _L)] = idx_v[p, pl.ds(c, _L)] + 1
            pltpu.async_copy(table_hbm.at[idx_v.at[p]], acc_v.at[p],
                             s_g.at[p], add=True)

        def drain_out(i, p):
            off = base + i * _C
            pltpu.make_async_copy(table_hbm.at[idx_v.at[p]], acc_v.at[p],
                                  s_g.at[p]).wait()
            pltpu.async_copy(acc_v.at[p], o_hbm.at[pl.ds(off, _C)], s_o.at[p])

        def wait_out(p):
            pltpu.make_async_copy(acc_v.at[p], o_hbm.at[pl.ds(0, _C)],
                                  s_o.at[p]).wait()

        start_inputs(0, 0)
        start_inputs(1, 1)
        fire_gather(0, 0)

        @pl.loop(0, n_chunks, step=_R)
        def _(i0):
            for j in range(_R):
                i = i0 + j
                pa = (j + 2) % _R
                pb = (j + 1) % _R

                @pl.when(i + 2 < n_chunks)
                def _():
                    @pl.when(i + 2 >= _R)
                    def _():
                        wait_out(pa)
                    start_inputs(i + 2, pa)

                @pl.when(i + 1 < n_chunks)
                def _():
                    fire_gather(i + 1, pb)

                drain_out(i, j)

        for p in range(_R):
            wait_out(p)

    v = run(table, idx, u2)
    return (z, v.reshape(B, S, D))
